# baseline (speedup 1.0000x reference)
import functools

import jax
import jax.numpy as jnp
from jax import lax
from jax.experimental import pallas as pl
from jax.experimental.pallas import tpu as pltpu
from jax.experimental.pallas import tpu_sc as plsc

K = 64
L = 8192
CHUNK = 128
ROWW = 128


def _digamma(x):
    acc = jnp.float32(0)
    for i in range(6):
        acc = acc + 1.0 / (x + jnp.float32(i))
    z = x + jnp.float32(6)
    zi = 1.0 / z
    zi2 = zi * zi
    psi = jnp.log(z) - 0.5 * zi - zi2 * (
        jnp.float32(1 / 12) - zi2 * (jnp.float32(1 / 120) - zi2 * jnp.float32(1 / 252))
    )
    return psi - acc


def _combine_body(b0_ref, b1_ref, g1_ref, g2_ref, w_ref, out_ref):
    b0 = b0_ref[...]
    b1 = b1_ref[...]
    d0 = _digamma(b0)
    d1 = _digamma(b1)
    dn = _digamma(b0 + b1)
    a = d0 - d1
    b = d1 - dn
    w = w_ref[...]
    mask = (w > 0).astype(jnp.float32)
    g1 = g1_ref[...][:, :K]
    g2 = g2_ref[...][:, :K]
    dims = (((0,), (0,)), ((), ()))
    m0 = lax.dot_general(g1 * mask, g2, dims, preferred_element_type=jnp.float32)
    mt = lax.dot_general(g1, g2, dims, preferred_element_type=jnp.float32)
    out_ref[...] = (jnp.sum(a * m0) + jnp.sum(b * mt)).reshape(1, 1)


def _transpose_body(x_ref, out_ref):
    x = x_ref[...]
    eye = jax.lax.broadcasted_iota(jnp.int32, (K, K), 0) == jax.lax.broadcasted_iota(
        jnp.int32, (K, K), 1
    )
    xt = lax.dot_general(
        x, eye.astype(jnp.float32), (((0,), (0,)), ((), ())),
        preferred_element_type=jnp.float32,
    )
    out_ref[...] = jnp.concatenate(
        [xt, jnp.zeros((x.shape[1], ROWW - K), jnp.float32)], axis=1
    )


TBLK = 512


def _build_table(eta_x, n_pad):
    nblk = n_pad // TBLK
    return pl.pallas_call(
        _transpose_body,
        grid=(nblk,),
        in_specs=[pl.BlockSpec((K, TBLK), lambda i: (0, i))],
        out_specs=pl.BlockSpec((TBLK, ROWW), lambda i: (i, 0)),
        out_shape=jax.ShapeDtypeStruct((n_pad, ROWW), jnp.float32),
    )(eta_x)


def _make_gather(nw, rows_per_w):
    nchunks = rows_per_w // CHUNK
    mesh = plsc.VectorSubcoreMesh(core_axis_name="c", subcore_axis_name="s")

    @functools.partial(
        pl.kernel,
        mesh=mesh,
        out_type=jax.ShapeDtypeStruct((nw, nchunks, CHUNK, ROWW), jnp.float32),
        scratch_types=[
            pltpu.VMEM((nchunks, CHUNK), jnp.int32),
            pltpu.VMEM((nchunks, CHUNK, ROWW), jnp.float32),
            pltpu.SemaphoreType.DMA,
        ],
        compiler_params=pltpu.CompilerParams(use_tc_tiling_on_sc=True),
    )
    def gather_kernel(idx_hbm, table_hbm, out_hbm, idx_v, rows_v, sem):
        nc = lax.axis_size("c")
        wid = lax.axis_index("s") * nc + lax.axis_index("c")
        pltpu.sync_copy(idx_hbm.at[wid], idx_v)
        copies = [
            pltpu.async_copy(table_hbm.at[idx_v.at[j]], rows_v.at[j], sem)
            for j in range(nchunks)
        ]
        for c in copies:
            c.wait()
        pltpu.sync_copy(rows_v, out_hbm.at[wid])

    return gather_kernel


def kernel(B_x, eta_x, idx1, idx2, weights):
    info = plsc.get_sparse_core_info()
    nw = info.num_cores * info.num_subcores
    n = eta_x.shape[1]
    n_pad = ((n + TBLK - 1) // TBLK) * TBLK
    table = _build_table(eta_x, n_pad)
    idx_all = jnp.concatenate([idx1, idx2]).astype(jnp.int32)
    rows_per_w = (2 * L) // nw
    idx_3d = idx_all.reshape(nw, rows_per_w // CHUNK, CHUNK)

    gathered = _make_gather(nw, rows_per_w)(idx_3d, table)
    rows = gathered.reshape(2 * L, ROWW)

    out = pl.pallas_call(
        _combine_body,
        grid=(1,),
        in_specs=[
            pl.BlockSpec((K, K), lambda i: (0, 0)),
            pl.BlockSpec((K, K), lambda i: (0, 0)),
            pl.BlockSpec((L, ROWW), lambda i: (0, 0)),
            pl.BlockSpec((L, ROWW), lambda i: (1, 0)),
            pl.BlockSpec((L, 1), lambda i: (0, 0)),
        ],
        out_specs=pl.BlockSpec((1, 1), lambda i: (0, 0)),
        out_shape=jax.ShapeDtypeStruct((1, 1), jnp.float32),
    )(B_x[:, :, 0], B_x[:, :, 1], rows, rows, weights.reshape(L, 1))
    return out[0, 0]


# speedup vs baseline: 1.4521x; 1.4521x over previous
import functools

import jax
import jax.numpy as jnp
from jax import lax
from jax.experimental import pallas as pl
from jax.experimental.pallas import tpu as pltpu
from jax.experimental.pallas import tpu_sc as plsc

K = 64
L = 8192
CHUNK = 128
ROWW = 128


def _digamma(x):
    acc = jnp.float32(0)
    for i in range(6):
        acc = acc + 1.0 / (x + jnp.float32(i))
    z = x + jnp.float32(6)
    zi = 1.0 / z
    zi2 = zi * zi
    psi = jnp.log(z) - 0.5 * zi - zi2 * (
        jnp.float32(1 / 12) - zi2 * (jnp.float32(1 / 120) - zi2 * jnp.float32(1 / 252))
    )
    return psi - acc


def _combine_body(b0_ref, b1_ref, g1_ref, g2_ref, w_ref, out_ref):
    b0 = b0_ref[...]
    b1 = b1_ref[...]
    d0 = _digamma(b0)
    d1 = _digamma(b1)
    dn = _digamma(b0 + b1)
    a = d0 - d1
    b = d1 - dn
    w = w_ref[...]
    mask = (w > 0).astype(jnp.float32)
    g1 = g1_ref[...][:, :K]
    g2 = g2_ref[...][:, :K]
    dims = (((0,), (0,)), ((), ()))
    m0 = lax.dot_general(g1 * mask, g2, dims, preferred_element_type=jnp.float32)
    mt = lax.dot_general(g1, g2, dims, preferred_element_type=jnp.float32)
    out_ref[...] = (jnp.sum(a * m0) + jnp.sum(b * mt)).reshape(1, 1)


def _make_gather(nw, rows_per_w):
    nchunks = rows_per_w // CHUNK
    mesh = plsc.VectorSubcoreMesh(core_axis_name="c", subcore_axis_name="s")

    @functools.partial(
        pl.kernel,
        mesh=mesh,
        out_type=jax.ShapeDtypeStruct((nw, nchunks, CHUNK, ROWW), jnp.float32),
        scratch_types=[
            pltpu.VMEM((nchunks, CHUNK), jnp.int32),
            pltpu.VMEM((nchunks, CHUNK, ROWW), jnp.float32),
            pltpu.SemaphoreType.DMA,
        ],
        compiler_params=pltpu.CompilerParams(use_tc_tiling_on_sc=True),
    )
    def gather_kernel(idx_hbm, table_hbm, out_hbm, idx_v, rows_v, sem):
        nc = lax.axis_size("c")
        wid = lax.axis_index("s") * nc + lax.axis_index("c")
        pltpu.sync_copy(idx_hbm.at[wid], idx_v)
        copies = [
            pltpu.async_copy(table_hbm.at[idx_v.at[j]], rows_v.at[j], sem)
            for j in range(nchunks)
        ]
        for c in copies:
            c.wait()
        pltpu.sync_copy(rows_v, out_hbm.at[wid])

    return gather_kernel


def kernel(B_x, eta_x, idx1, idx2, weights):
    info = plsc.get_sparse_core_info()
    nw = info.num_cores * info.num_subcores
    n = eta_x.shape[1]
    table = jnp.zeros((n, ROWW), jnp.float32).at[:, :K].set(eta_x.T)
    idx_all = jnp.concatenate([idx1, idx2]).astype(jnp.int32)
    rows_per_w = (2 * L) // nw
    idx_3d = idx_all.reshape(nw, rows_per_w // CHUNK, CHUNK)

    gathered = _make_gather(nw, rows_per_w)(idx_3d, table)
    rows = gathered.reshape(2 * L, ROWW)

    out = pl.pallas_call(
        _combine_body,
        grid=(1,),
        in_specs=[
            pl.BlockSpec((K, K), lambda i: (0, 0)),
            pl.BlockSpec((K, K), lambda i: (0, 0)),
            pl.BlockSpec((L, ROWW), lambda i: (0, 0)),
            pl.BlockSpec((L, ROWW), lambda i: (1, 0)),
            pl.BlockSpec((L, 1), lambda i: (0, 0)),
        ],
        out_specs=pl.BlockSpec((1, 1), lambda i: (0, 0)),
        out_shape=jax.ShapeDtypeStruct((1, 1), jnp.float32),
    )(B_x[:, :, 0], B_x[:, :, 1], rows, rows, weights.reshape(L, 1))
    return out[0, 0]


# speedup vs baseline: 1.4589x; 1.0047x over previous
import functools

import jax
import jax.numpy as jnp
from jax import lax
from jax.experimental import pallas as pl
from jax.experimental.pallas import tpu as pltpu
from jax.experimental.pallas import tpu_sc as plsc

K = 64
L = 8192
CHUNK = 128
ROWW = 128


def _digamma(x):
    acc = jnp.float32(0)
    for i in range(6):
        acc = acc + 1.0 / (x + jnp.float32(i))
    z = x + jnp.float32(6)
    zi = 1.0 / z
    zi2 = zi * zi
    psi = jnp.log(z) - 0.5 * zi - zi2 * (
        jnp.float32(1 / 12) - zi2 * (jnp.float32(1 / 120) - zi2 * jnp.float32(1 / 252))
    )
    return psi - acc


def _combine_body(b0_ref, b1_ref, g1_ref, g2_ref, w_ref, out_ref):
    b0 = b0_ref[...]
    b1 = b1_ref[...]
    d0 = _digamma(b0)
    d1 = _digamma(b1)
    dn = _digamma(b0 + b1)
    a = d0 - d1
    b = d1 - dn
    w = w_ref[...]
    mask = (w > 0).astype(jnp.float32)
    g1 = g1_ref[...][:, :K]
    g2 = g2_ref[...][:, :K]
    dims = (((0,), (0,)), ((), ()))
    m0 = lax.dot_general(g1 * mask, g2, dims, preferred_element_type=jnp.float32)
    mt = lax.dot_general(g1, g2, dims, preferred_element_type=jnp.float32)
    out_ref[...] = (jnp.sum(a * m0) + jnp.sum(b * mt)).reshape(1, 1)


def _make_gather(nw, rows_per_w):
    mesh = plsc.VectorSubcoreMesh(core_axis_name="c", subcore_axis_name="s")

    @functools.partial(
        pl.kernel,
        mesh=mesh,
        out_type=jax.ShapeDtypeStruct((nw, rows_per_w, ROWW), jnp.float32),
        scratch_types=[
            pltpu.VMEM((rows_per_w,), jnp.int32),
            pltpu.VMEM((rows_per_w, ROWW), jnp.float32),
            pltpu.SemaphoreType.DMA,
        ],
        compiler_params=pltpu.CompilerParams(use_tc_tiling_on_sc=True),
    )
    def gather_kernel(idx_hbm, table_hbm, out_hbm, idx_v, rows_v, sem):
        nc = lax.axis_size("c")
        wid = lax.axis_index("s") * nc + lax.axis_index("c")
        pltpu.sync_copy(idx_hbm.at[wid], idx_v)
        pltpu.async_copy(table_hbm.at[idx_v], rows_v, sem).wait()
        pltpu.sync_copy(rows_v, out_hbm.at[wid])

    return gather_kernel


def kernel(B_x, eta_x, idx1, idx2, weights):
    info = plsc.get_sparse_core_info()
    nw = info.num_cores * info.num_subcores
    n = eta_x.shape[1]
    table = jnp.zeros((n, ROWW), jnp.float32).at[:, :K].set(eta_x.T)
    idx_all = jnp.concatenate([idx1, idx2]).astype(jnp.int32)
    rows_per_w = (2 * L) // nw
    idx_2d = idx_all.reshape(nw, rows_per_w)

    gathered = _make_gather(nw, rows_per_w)(idx_2d, table)
    rows = gathered.reshape(2 * L, ROWW)

    out = pl.pallas_call(
        _combine_body,
        grid=(1,),
        in_specs=[
            pl.BlockSpec((K, K), lambda i: (0, 0)),
            pl.BlockSpec((K, K), lambda i: (0, 0)),
            pl.BlockSpec((L, ROWW), lambda i: (0, 0)),
            pl.BlockSpec((L, ROWW), lambda i: (1, 0)),
            pl.BlockSpec((L, 1), lambda i: (0, 0)),
        ],
        out_specs=pl.BlockSpec((1, 1), lambda i: (0, 0)),
        out_shape=jax.ShapeDtypeStruct((1, 1), jnp.float32),
    )(B_x[:, :, 0], B_x[:, :, 1], rows, rows, weights.reshape(L, 1))
    return out[0, 0]


# speedup vs baseline: 1.4614x; 1.0017x over previous
import functools

import jax
import jax.numpy as jnp
from jax import lax
from jax.experimental import pallas as pl
from jax.experimental.pallas import tpu as pltpu
from jax.experimental.pallas import tpu_sc as plsc

K = 64
L = 8192
ROWW = 128


def _digamma(x):
    acc = jnp.float32(0)
    for i in range(6):
        acc = acc + 1.0 / (x + jnp.float32(i))
    z = x + jnp.float32(6)
    zi = 1.0 / z
    zi2 = zi * zi
    psi = jnp.log(z) - 0.5 * zi - zi2 * (
        jnp.float32(1 / 12) - zi2 * (jnp.float32(1 / 120) - zi2 * jnp.float32(1 / 252))
    )
    return psi - acc


def _combine_body(b0_ref, b1_ref, g1_ref, g2_ref, w_ref, out_ref):
    b0 = b0_ref[...]
    b1 = b1_ref[...]
    d0 = _digamma(b0)
    d1 = _digamma(b1)
    dn = _digamma(b0 + b1)
    a = d0 - d1
    b = d1 - dn
    w = w_ref[...]
    mask = (w > 0).astype(jnp.float32)
    g1 = g1_ref[...][:, :K]
    g2 = g2_ref[...][:, :K]
    dims = (((0,), (0,)), ((), ()))
    m0 = lax.dot_general(g1 * mask, g2, dims, preferred_element_type=jnp.float32)
    mt = lax.dot_general(g1, g2, dims, preferred_element_type=jnp.float32)
    out_ref[...] = (jnp.sum(a * m0) + jnp.sum(b * mt)).reshape(1, 1)


def _make_gather(nw, rows_per_w):
    mesh = plsc.VectorSubcoreMesh(core_axis_name="c", subcore_axis_name="s")

    @functools.partial(
        pl.kernel,
        mesh=mesh,
        out_type=jax.ShapeDtypeStruct((nw, rows_per_w, ROWW), jnp.float32),
        scratch_types=[
            pltpu.VMEM((rows_per_w,), jnp.int32),
            pltpu.VMEM((rows_per_w, ROWW), jnp.float32),
            pltpu.SemaphoreType.DMA,
        ],
        compiler_params=pltpu.CompilerParams(use_tc_tiling_on_sc=True),
    )
    def gather_kernel(idx_hbm, table_hbm, out_hbm, idx_v, rows_v, sem):
        nc = lax.axis_size("c")
        wid = lax.axis_index("s") * nc + lax.axis_index("c")
        pltpu.sync_copy(idx_hbm.at[wid], idx_v)
        pltpu.async_copy(table_hbm.at[idx_v], rows_v, sem).wait()
        pltpu.sync_copy(rows_v, out_hbm.at[wid])

    return gather_kernel


def kernel(B_x, eta_x, idx1, idx2, weights):
    info = plsc.get_sparse_core_info()
    nw = info.num_cores * info.num_subcores
    n = eta_x.shape[1]
    table = jnp.zeros((n, ROWW), jnp.float32).at[:, :K].set(eta_x.T)
    idx_all = jnp.concatenate([idx1, idx2]).astype(jnp.int32)
    rows_per_w = (2 * L) // nw
    idx_2d = idx_all.reshape(nw, rows_per_w)

    gathered = _make_gather(nw, rows_per_w)(idx_2d, table)
    rows = gathered.reshape(2 * L, ROWW)

    out = pl.pallas_call(
        _combine_body,
        grid=(1,),
        in_specs=[
            pl.BlockSpec((K, K), lambda i: (0, 0)),
            pl.BlockSpec((K, K), lambda i: (0, 0)),
            pl.BlockSpec((L, ROWW), lambda i: (0, 0)),
            pl.BlockSpec((L, ROWW), lambda i: (1, 0)),
            pl.BlockSpec((L, 1), lambda i: (0, 0)),
        ],
        out_specs=pl.BlockSpec((1, 1), lambda i: (0, 0)),
        out_shape=jax.ShapeDtypeStruct((1, 1), jnp.float32),
    )(B_x[:, :, 0], B_x[:, :, 1], rows, rows, weights.reshape(L, 1))
    return out[0, 0]
